# Initial kernel scaffold; baseline (speedup 1.0000x reference)
import jax
import jax.numpy as jnp
from jax.experimental import pallas as pl


def kernel(x, edge_index, W1, b1, Wq, bq, Wk, bk, Wv, bv, Ws, bs, W2, b2):
    raise NotImplementedError("write your pallas kernel here")


# speedup vs baseline: 12.0458x; 12.0458x over previous
import functools

import jax
import jax.numpy as jnp
from jax import lax
from jax.experimental import pallas as pl
from jax.experimental.pallas import tpu as pltpu
from jax.experimental.pallas import tpu_sc as plsc

N = 10000
E = 320000
DI = 128
DH = 64
DO = 128

NC = 2
NS = 16
NW = NC * NS
EPW = E // NW
C = 80
NCH = EPW // C
R0 = 624
RL = N - R0 * (NS - 1)

BN = 1000
GRID = N // BN

_mesh = plsc.VectorSubcoreMesh(
    core_axis_name="c", subcore_axis_name="s", num_cores=NC, num_subcores=NS
)
f32 = jnp.float32


def _wids():
    cid = lax.axis_index("c")
    sid = lax.axis_index("s")
    return cid, sid, sid * NC + cid


def _copy_rows(src, dst, sid):
    @pl.when(sid < NS - 1)
    def _():
        pltpu.sync_copy(src.at[pl.ds(sid * R0, R0)], dst.at[pl.ds(sid * R0, R0)])

    @pl.when(sid == NS - 1)
    def _():
        pltpu.sync_copy(
            src.at[pl.ds((NS - 1) * R0, RL)], dst.at[pl.ds((NS - 1) * R0, RL)]
        )


@functools.partial(
    pl.kernel,
    out_type=jax.ShapeDtypeStruct((NC, N), f32),
    mesh=_mesh,
    compiler_params=pltpu.CompilerParams(use_tc_tiling_on_sc=False, needs_layout_passes=False),
    scratch_types=[
        pltpu.VMEM((NCH, C), jnp.int32),
        pltpu.VMEM((C,), f32),
        pltpu.VMEM_SHARED((N,), f32),
    ],
)
def _deg_sc(dstr, zeros1, out, didx, onesv, acc):
    cid, sid, wid = _wids()
    pltpu.sync_copy(dstr.at[wid], didx)
    for k in range(C // 16):
        onesv[pl.ds(k * 16, 16)] = jnp.ones((16,), f32)

    @pl.when(sid == 0)
    def _():
        pltpu.sync_copy(zeros1, acc)

    plsc.subcore_barrier()

    def body(j, carry):
        pltpu.sync_copy(onesv, acc.at[didx.at[j]], add=True)
        return carry

    lax.fori_loop(0, NCH, body, 0)
    plsc.subcore_barrier()

    @pl.when(sid == 0)
    def _():
        pltpu.sync_copy(acc, out.at[cid])


def _make_gs_sc(W):
    @functools.partial(
        pl.kernel,
        out_type=jax.ShapeDtypeStruct((NC, N, W), f32),
        mesh=_mesh,
    compiler_params=pltpu.CompilerParams(use_tc_tiling_on_sc=False, needs_layout_passes=False),
        scratch_types=[
            pltpu.VMEM((NCH, C), jnp.int32),
            pltpu.VMEM((NCH, C), jnp.int32),
            pltpu.VMEM((C, W), f32),
            pltpu.VMEM((C, W), f32),
            pltpu.SemaphoreType.DMA,
            pltpu.SemaphoreType.DMA,
            pltpu.VMEM_SHARED((N, W), f32),
        ],
    )
    def gs(y, srcr, dstr, zz, out, sidx, didx, buf0, buf1, sem0, sem1, acc):
        cid, sid, wid = _wids()
        pltpu.sync_copy(srcr.at[wid], sidx)
        pltpu.sync_copy(dstr.at[wid], didx)
        _copy_rows(zz, acc, sid)
        plsc.subcore_barrier()

        bufs = (buf0, buf1)
        sems = (sem0, sem1)
        pltpu.async_copy(y.at[sidx.at[0]], buf0, sem0)

        def body(g, carry):
            for b in range(2):
                j = g * 2 + b
                cur, csem = bufs[b], sems[b]
                nxt, nsem = bufs[1 - b], sems[1 - b]
                pltpu.async_copy(y.at[sidx.at[j + 1]], nxt, nsem)
                pltpu.make_async_copy(y.at[sidx.at[j]], cur, csem).wait()
                pltpu.sync_copy(cur, acc.at[didx.at[j]], add=True)
            return carry

        lax.fori_loop(0, (NCH - 1) // 2, body, 0)
        pltpu.make_async_copy(y.at[sidx.at[NCH - 1]], buf0, sem0).wait()
        pltpu.sync_copy(buf0, acc.at[didx.at[NCH - 1]], add=True)

        plsc.subcore_barrier()
        _copy_rows(acc, out.at[cid], sid)

    return gs


_gs64_sc = _make_gs_sc(DH)
_gs128_sc = _make_gs_sc(DO)


@functools.partial(
    pl.kernel,
    out_type=jax.ShapeDtypeStruct((NW, NCH, C), f32),
    mesh=_mesh,
    compiler_params=pltpu.CompilerParams(use_tc_tiling_on_sc=False, needs_layout_passes=False),
    scratch_types=[
        pltpu.VMEM((NCH, C), jnp.int32),
        pltpu.VMEM((NCH, C), jnp.int32),
        pltpu.VMEM((C, DH), f32),
        pltpu.VMEM((C, DH), f32),
        pltpu.VMEM((NCH, C), f32),
        pltpu.SemaphoreType.DMA,
        pltpu.SemaphoreType.DMA,
    ],
)
def _alpha_sc(q, k, srcr, dstr, aout, sidx, didx, qb, kb, abuf, semq, semk):
    cid, sid, wid = _wids()
    pltpu.sync_copy(srcr.at[wid], sidx)
    pltpu.sync_copy(dstr.at[wid], didx)

    def chunk(j, carry):
        dq = pltpu.async_copy(q.at[didx.at[j]], qb, semq)
        dk = pltpu.async_copy(k.at[sidx.at[j]], kb, semk)
        dq.wait()
        dk.wait()
        for v in range(C // 16):
            rowv = lax.iota(jnp.int32, 16) + (v * 16)

            def colb(cb, acc):
                for cc in range(4):
                    col = cb * 4 + cc
                    colv = jnp.zeros((16,), jnp.int32) + col
                    qv = plsc.load_gather(qb, [rowv, colv])
                    kv = plsc.load_gather(kb, [rowv, colv])
                    acc = acc + qv * kv
                return acc

            accv = lax.fori_loop(0, DH // 4, colb, jnp.zeros((16,), f32))
            abuf[j, pl.ds(v * 16, 16)] = accv * 0.125
        return carry

    lax.fori_loop(0, NCH, chunk, 0)
    pltpu.sync_copy(abuf, aout.at[wid])


@functools.partial(
    pl.kernel,
    out_type=[
        jax.ShapeDtypeStruct((NC, N, DH), f32),
        jax.ShapeDtypeStruct((NC, N), f32),
    ],
    mesh=_mesh,
    compiler_params=pltpu.CompilerParams(use_tc_tiling_on_sc=False, needs_layout_passes=False),
    scratch_types=[
        pltpu.VMEM((NCH, C), jnp.int32),
        pltpu.VMEM((NCH, C), jnp.int32),
        pltpu.VMEM((NCH, C), f32),
        pltpu.VMEM((NCH, C), f32),
        pltpu.VMEM((C, DH), f32),
        pltpu.VMEM((16,), f32),
        pltpu.SemaphoreType.DMA,
        pltpu.VMEM_SHARED((N, DH), f32),
        pltpu.VMEM_SHARED((N,), f32),
    ],
)
def _attn_sc(v_t, alpha, m16, srcr, dstr, z64, z1, num, den,
             sidx, didx, abuf, ebuf, vbuf, mref, sem, accn, accd):
    cid, sid, wid = _wids()
    pltpu.sync_copy(srcr.at[wid], sidx)
    pltpu.sync_copy(dstr.at[wid], didx)
    pltpu.sync_copy(alpha.at[wid], abuf)
    pltpu.sync_copy(m16, mref)
    _copy_rows(z64, accn, sid)

    @pl.when(sid == 0)
    def _():
        pltpu.sync_copy(z1, accd)

    plsc.subcore_barrier()
    mv = mref[...]

    def exbody(j, carry):
        for kk in range(C // 16):
            a = abuf[j, pl.ds(kk * 16, 16)]
            ebuf[j, pl.ds(kk * 16, 16)] = jnp.exp(a - mv)
        return carry

    lax.fori_loop(0, NCH, exbody, 0)

    def chunk(j, carry):
        pltpu.async_copy(v_t.at[sidx.at[j]], vbuf, sem).wait()

        def scale_grp(g, c2):
            ex16 = ebuf[j, pl.ds(g * 16, 16)]
            base = g * 16
            for l in range(16):
                s = ex16[l]
                for k4 in range(DH // 16):
                    vbuf[base + l, pl.ds(k4 * 16, 16)] = (
                        vbuf[base + l, pl.ds(k4 * 16, 16)] * s
                    )
            return c2

        lax.fori_loop(0, C // 16, scale_grp, 0)
        pltpu.sync_copy(vbuf, accn.at[didx.at[j]], add=True)
        pltpu.sync_copy(ebuf.at[j], accd.at[didx.at[j]], add=True)
        return carry

    lax.fori_loop(0, NCH, chunk, 0)
    plsc.subcore_barrier()
    _copy_rows(accn, num.at[cid], sid)

    @pl.when(sid == 0)
    def _():
        pltpu.sync_copy(accd, den.at[cid])


def _t1_body(degp, x, w1, dinv_o, y1_o):
    deg = degp[0] + degp[1] + 1.0
    di = lax.rsqrt(deg)
    xw = jnp.dot(x[...], w1[...], preferred_element_type=f32)
    dinv_o[...] = di
    y1_o[...] = xw * di


def _t1(degp, x, w1):
    return pl.pallas_call(
        _t1_body,
        grid=(GRID,),
        in_specs=[
            pl.BlockSpec((NC, BN, 1), lambda i: (0, i, 0)),
            pl.BlockSpec((BN, DI), lambda i: (i, 0)),
            pl.BlockSpec((DI, DH), lambda i: (0, 0)),
        ],
        out_specs=[
            pl.BlockSpec((BN, 1), lambda i: (i, 0)),
            pl.BlockSpec((BN, DH), lambda i: (i, 0)),
        ],
        out_shape=[
            jax.ShapeDtypeStruct((N, 1), f32),
            jax.ShapeDtypeStruct((N, DH), f32),
        ],
    )(degp, x, w1)


def _t2_body(p, y1, dinv, b1, wq, bq, wk, bk, wv, bv, ws, bs,
             q_o, k_o, v_o, skip_o):
    di = dinv[...]
    h = jnp.maximum(di * (p[0] + p[1] + y1[...]) + b1[...], 0.0)
    q_o[...] = jnp.dot(h, wq[...], preferred_element_type=f32) + bq[...]
    k_o[...] = jnp.dot(h, wk[...], preferred_element_type=f32) + bk[...]
    v_o[...] = jnp.dot(h, wv[...], preferred_element_type=f32) + bv[...]
    skip_o[...] = jnp.dot(h, ws[...], preferred_element_type=f32) + bs[...]


def _t2(p, y1, dinv, b1, wq, bq, wk, bk, wv, bv, ws, bs):
    wspec = pl.BlockSpec((DH, DH), lambda i: (0, 0))
    bspec = pl.BlockSpec((1, DH), lambda i: (0, 0))
    nspec = pl.BlockSpec((BN, DH), lambda i: (i, 0))
    return pl.pallas_call(
        _t2_body,
        grid=(GRID,),
        in_specs=[
            pl.BlockSpec((NC, BN, DH), lambda i: (0, i, 0)),
            nspec,
            pl.BlockSpec((BN, 1), lambda i: (i, 0)),
            bspec, wspec, bspec, wspec, bspec, wspec, bspec, wspec, bspec,
        ],
        out_specs=[nspec, nspec, nspec, nspec],
        out_shape=[jax.ShapeDtypeStruct((N, DH), f32)] * 4,
    )(p, y1, dinv, b1, wq, bq, wk, bk, wv, bv, ws, bs)


def _t3_body(a, m_o):
    m_o[...] = jnp.max(a[...]).reshape(1, 1)


def _t3(alpha2d):
    return pl.pallas_call(
        _t3_body,
        out_shape=jax.ShapeDtypeStruct((1, 1), f32),
    )(alpha2d)


def _t4_body(num, den, skip, dinv, w2, y2_o):
    d = den[0] + den[1]
    h2 = (num[0] + num[1]) / (d + 1e-16) + skip[...]
    xw2 = jnp.dot(h2, w2[...], preferred_element_type=f32)
    y2_o[...] = xw2 * dinv[...]


def _t4(num, den, skip, dinv, w2):
    return pl.pallas_call(
        _t4_body,
        grid=(GRID,),
        in_specs=[
            pl.BlockSpec((NC, BN, DH), lambda i: (0, i, 0)),
            pl.BlockSpec((NC, BN, 1), lambda i: (0, i, 0)),
            pl.BlockSpec((BN, DH), lambda i: (i, 0)),
            pl.BlockSpec((BN, 1), lambda i: (i, 0)),
            pl.BlockSpec((DH, DO), lambda i: (0, 0)),
        ],
        out_specs=pl.BlockSpec((BN, DO), lambda i: (i, 0)),
        out_shape=jax.ShapeDtypeStruct((N, DO), f32),
    )(num, den, skip, dinv, w2)


def _t5_body(p, y2, dinv, b2, out_o):
    out_o[...] = dinv[...] * (p[0] + p[1] + y2[...]) + b2[...]


def _t5(p, y2, dinv, b2):
    return pl.pallas_call(
        _t5_body,
        grid=(GRID,),
        in_specs=[
            pl.BlockSpec((NC, BN, DO), lambda i: (0, i, 0)),
            pl.BlockSpec((BN, DO), lambda i: (i, 0)),
            pl.BlockSpec((BN, 1), lambda i: (i, 0)),
            pl.BlockSpec((1, DO), lambda i: (0, 0)),
        ],
        out_specs=pl.BlockSpec((BN, DO), lambda i: (i, 0)),
        out_shape=jax.ShapeDtypeStruct((N, DO), f32),
    )(p, y2, dinv, b2)


def kernel(x, edge_index, W1, b1, Wq, bq, Wk, bk, Wv, bv, Ws, bs, W2, b2):
    src = edge_index[0].reshape(NW, NCH, C)
    dst = edge_index[1].reshape(NW, NCH, C)
    zeros1 = jnp.zeros((N,), f32)
    zeros64 = jnp.zeros((N, DH), f32)
    zeros128 = jnp.zeros((N, DO), f32)

    degp = _deg_sc(dst, zeros1)
    dinv, y1 = _t1(degp.reshape(NC, N, 1), x, W1)
    p1 = _gs64_sc(y1, src, dst, zeros64)
    q, k, v, skip = _t2(
        p1, y1, dinv, b1.reshape(1, DH),
        Wq, bq.reshape(1, DH), Wk, bk.reshape(1, DH),
        Wv, bv.reshape(1, DH), Ws, bs.reshape(1, DH),
    )
    alpha = _alpha_sc(q, k, src, dst)
    m = _t3(alpha.reshape(E // 128, 128))
    m16 = jnp.broadcast_to(m[0, 0], (16,))
    num, den = _attn_sc(v, alpha, m16, src, dst, zeros64, zeros1)
    y2 = _t4(num, den.reshape(NC, N, 1), skip, dinv, W2)
    p3 = _gs128_sc(y2, src, dst, zeros128)
    return _t5(p3, y2, dinv, b2.reshape(1, DO))
